# baseline (speedup 1.0000x reference)
import jax
import jax.numpy as jnp
from jax.experimental import pallas as pl
from jax.experimental.pallas import tpu as pltpu

_BLK = 8192
_CHK = 512
_EPS = 1e-5


def _ln_mod(h, mask6, lnw, lnb, g, bb):
    mu = jnp.sum(h, axis=0, keepdims=True) * (1.0 / 6.0)
    d = (h - mu) * mask6
    var = jnp.sum(d * d, axis=0, keepdims=True) * (1.0 / 6.0)
    hn = d * jax.lax.rsqrt(var + _EPS)
    return g * (hn * lnw + lnb) + bb


def _body(xyT_ref, tab_ref, prm_ref, whm_ref, wout_ref, out_ref, itp_ref):
    prm = prm_ref[...]
    w_in_x = prm[:, 0:1]
    w_in_y = prm[:, 1:2]
    b_in = prm[:, 2:3]
    lnw0 = prm[:, 3:4]
    lnb0 = prm[:, 4:5]
    lnw1 = prm[:, 5:6]
    lnb1 = prm[:, 6:7]
    b_h = prm[:, 7:8]
    b_out = prm[:, 8:9]
    mask6 = prm[:, 9:10]
    tab = tab_ref[...]
    iot = jax.lax.broadcasted_iota(jnp.int32, (16, _CHK), 0).astype(jnp.float32)

    for j in range(_BLK // _CHK):
        lo, hi = j * _CHK, (j + 1) * _CHK
        xy = xyT_ref[:, lo:hi]
        cx = jnp.clip((xy[0:1, :] + 1.0) * 7.5, 0.0, 15.0)
        cy = jnp.clip((xy[1:2, :] + 1.0) * 7.5, 0.0, 15.0)
        ohx = jnp.maximum(0.0, 1.0 - jnp.abs(cx - iot))
        ohy = jnp.maximum(0.0, 1.0 - jnp.abs(cy - iot))
        w_interp = (ohy[:, None, :] * ohx[None, :, :]).reshape(256, _CHK)
        r = jnp.dot(tab, w_interp.astype(jnp.bfloat16),
                    preferred_element_type=jnp.float32)
        itp_ref[:, lo:hi] = r[0:32, :] + r[32:64, :]

    x = xyT_ref[0:1, :]
    y = xyT_ref[1:2, :]
    itp = itp_ref[...]
    g0 = itp[0:8, :]
    bb0 = itp[8:16, :]
    g1 = itp[16:24, :]
    bb1 = itp[24:32, :]
    h = x * w_in_x + y * w_in_y + b_in
    h = h * jax.nn.sigmoid(h)
    h = _ln_mod(h, mask6, lnw0, lnb0, g0, bb0)
    z = jnp.dot(whm_ref[...], h, preferred_element_type=jnp.float32,
                precision=jax.lax.Precision.HIGHEST) + b_h
    h = z * jax.nn.sigmoid(z)
    h = _ln_mod(h, mask6, lnw1, lnb1, g1, bb1)
    o = jnp.dot(wout_ref[...], h, preferred_element_type=jnp.float32,
                precision=jax.lax.Precision.HIGHEST) + b_out
    out_ref[...] = o[0:3, :]


def kernel(xy, gamma, beta, w_in, b_in, w_h, b_h, w_out, b_out, ln_w, ln_b):
    n = xy.shape[0]
    xyT = xy.T
    t = jnp.zeros((32, 256), jnp.float32)
    t = t.at[0:6].set(gamma[0].reshape(6, 256))
    t = t.at[8:14].set(beta[0].reshape(6, 256))
    t = t.at[16:22].set(gamma[1].reshape(6, 256))
    t = t.at[24:30].set(beta[1].reshape(6, 256))
    t_hi = t.astype(jnp.bfloat16)
    t_lo = (t - t_hi.astype(jnp.float32)).astype(jnp.bfloat16)
    tab = jnp.concatenate([t_hi, t_lo], axis=0)

    def col(v):
        return jnp.pad(v.astype(jnp.float32), (0, 8 - v.shape[0]))

    prm = jnp.stack([
        col(w_in[:, 0]), col(w_in[:, 1]), col(b_in),
        col(ln_w[0]), col(ln_b[0]), col(ln_w[1]), col(ln_b[1]),
        col(b_h[0]), col(b_out),
        jnp.array([1, 1, 1, 1, 1, 1, 0, 0], jnp.float32),
    ] + [jnp.zeros(8, jnp.float32)] * 6, axis=1)
    whm = jnp.zeros((8, 8), jnp.float32).at[0:6, 0:6].set(w_h[0])
    wout = jnp.zeros((8, 8), jnp.float32).at[0:3, 0:6].set(w_out)

    outT = pl.pallas_call(
        _body,
        out_shape=jax.ShapeDtypeStruct((3, n), jnp.float32),
        grid=(n // _BLK,),
        in_specs=[
            pl.BlockSpec((2, _BLK), lambda i: (0, i)),
            pl.BlockSpec((64, 256), lambda i: (0, 0)),
            pl.BlockSpec((8, 16), lambda i: (0, 0)),
            pl.BlockSpec((8, 8), lambda i: (0, 0)),
            pl.BlockSpec((8, 8), lambda i: (0, 0)),
        ],
        out_specs=pl.BlockSpec((3, _BLK), lambda i: (0, i)),
        scratch_shapes=[pltpu.VMEM((32, _BLK), jnp.float32)],
        compiler_params=pltpu.CompilerParams(
            dimension_semantics=("arbitrary",),
        ),
        name="camfield_fused",
    )(xyT, tab, prm, whm, wout)
    return outT.T


# speedup vs baseline: 1.4042x; 1.4042x over previous
import jax
import jax.numpy as jnp
from jax.experimental import pallas as pl
from jax.experimental.pallas import tpu as pltpu

_BLK = 16384
_CHK = 512
_EPS = 1e-5


def _ln_mod(h, mask6, lnw, lnb, g, bb):
    mu = jnp.sum(h, axis=0, keepdims=True) * (1.0 / 6.0)
    d = (h - mu) * mask6
    var = jnp.sum(d * d, axis=0, keepdims=True) * (1.0 / 6.0)
    hn = d * jax.lax.rsqrt(var + _EPS)
    return g * (hn * lnw + lnb) + bb


def _body(xyT_ref, tab_ref, prm_ref, whm_ref, wout_ref, out_ref, itp_ref):
    prm = prm_ref[...]
    w_in_x = prm[:, 0:1]
    w_in_y = prm[:, 1:2]
    b_in = prm[:, 2:3]
    lnw0 = prm[:, 3:4]
    lnb0 = prm[:, 4:5]
    lnw1 = prm[:, 5:6]
    lnb1 = prm[:, 6:7]
    b_h = prm[:, 7:8]
    b_out = prm[:, 8:9]
    mask6 = prm[:, 9:10]
    tab = tab_ref[...]
    iot = jax.lax.broadcasted_iota(jnp.int32, (16, _CHK), 0).astype(jnp.float32)

    for j in range(_BLK // _CHK):
        lo, hi = j * _CHK, (j + 1) * _CHK
        xy = xyT_ref[:, lo:hi]
        cx = jnp.clip((xy[0:1, :] + 1.0) * 7.5, 0.0, 15.0)
        cy = jnp.clip((xy[1:2, :] + 1.0) * 7.5, 0.0, 15.0)
        ohx = jnp.maximum(0.0, 1.0 - jnp.abs(cx - iot))
        ohy = jnp.maximum(0.0, 1.0 - jnp.abs(cy - iot))
        w_interp = (ohy[:, None, :] * ohx[None, :, :]).reshape(256, _CHK)
        r = jnp.dot(tab, w_interp.astype(jnp.bfloat16),
                    preferred_element_type=jnp.float32)
        itp_ref[:, lo:hi] = r[0:32, :] + r[32:64, :]

    x = xyT_ref[0:1, :]
    y = xyT_ref[1:2, :]
    itp = itp_ref[...]
    g0 = itp[0:8, :]
    bb0 = itp[8:16, :]
    g1 = itp[16:24, :]
    bb1 = itp[24:32, :]
    h = x * w_in_x + y * w_in_y + b_in
    h = h * jax.nn.sigmoid(h)
    h = _ln_mod(h, mask6, lnw0, lnb0, g0, bb0)
    z = jnp.dot(whm_ref[...], h, preferred_element_type=jnp.float32) + b_h
    h = z * jax.nn.sigmoid(z)
    h = _ln_mod(h, mask6, lnw1, lnb1, g1, bb1)
    o = jnp.dot(wout_ref[...], h, preferred_element_type=jnp.float32) + b_out
    out_ref[...] = o[0:3, :]


def kernel(xy, gamma, beta, w_in, b_in, w_h, b_h, w_out, b_out, ln_w, ln_b):
    n = xy.shape[0]
    xyT = xy.T
    t = jnp.zeros((32, 256), jnp.float32)
    t = t.at[0:6].set(gamma[0].reshape(6, 256))
    t = t.at[8:14].set(beta[0].reshape(6, 256))
    t = t.at[16:22].set(gamma[1].reshape(6, 256))
    t = t.at[24:30].set(beta[1].reshape(6, 256))
    t_hi = t.astype(jnp.bfloat16)
    t_lo = (t - t_hi.astype(jnp.float32)).astype(jnp.bfloat16)
    tab = jnp.concatenate([t_hi, t_lo], axis=0)

    def col(v):
        return jnp.pad(v.astype(jnp.float32), (0, 8 - v.shape[0]))

    prm = jnp.stack([
        col(w_in[:, 0]), col(w_in[:, 1]), col(b_in),
        col(ln_w[0]), col(ln_b[0]), col(ln_w[1]), col(ln_b[1]),
        col(b_h[0]), col(b_out),
        jnp.array([1, 1, 1, 1, 1, 1, 0, 0], jnp.float32),
    ] + [jnp.zeros(8, jnp.float32)] * 6, axis=1)
    whm = jnp.zeros((8, 8), jnp.float32).at[0:6, 0:6].set(w_h[0])
    wout = jnp.zeros((8, 8), jnp.float32).at[0:3, 0:6].set(w_out)

    outT = pl.pallas_call(
        _body,
        out_shape=jax.ShapeDtypeStruct((3, n), jnp.float32),
        grid=(n // _BLK,),
        in_specs=[
            pl.BlockSpec((2, _BLK), lambda i: (0, i)),
            pl.BlockSpec((64, 256), lambda i: (0, 0)),
            pl.BlockSpec((8, 16), lambda i: (0, 0)),
            pl.BlockSpec((8, 8), lambda i: (0, 0)),
            pl.BlockSpec((8, 8), lambda i: (0, 0)),
        ],
        out_specs=pl.BlockSpec((3, _BLK), lambda i: (0, i)),
        scratch_shapes=[pltpu.VMEM((32, _BLK), jnp.float32)],
        compiler_params=pltpu.CompilerParams(
            dimension_semantics=("arbitrary",),
        ),
        name="camfield_fused",
    )(xyT, tab, prm, whm, wout)
    return outT.T


# speedup vs baseline: 1.4492x; 1.0320x over previous
import jax
import jax.numpy as jnp
from jax.experimental import pallas as pl
from jax.experimental.pallas import tpu as pltpu

_BLK = 32768
_CHK = 512
_EPS = 1e-5


def _ln_mod(h, mask6, lnw, lnb, g, bb):
    mu = jnp.sum(h, axis=0, keepdims=True) * (1.0 / 6.0)
    d = (h - mu) * mask6
    var = jnp.sum(d * d, axis=0, keepdims=True) * (1.0 / 6.0)
    hn = d * jax.lax.rsqrt(var + _EPS)
    return g * (hn * lnw + lnb) + bb


def _body(xyT_ref, tab_ref, prm_ref, whm_ref, wout_ref, out_ref, itp_ref):
    prm = prm_ref[...]
    w_in_x = prm[:, 0:1]
    w_in_y = prm[:, 1:2]
    b_in = prm[:, 2:3]
    lnw0 = prm[:, 3:4]
    lnb0 = prm[:, 4:5]
    lnw1 = prm[:, 5:6]
    lnb1 = prm[:, 6:7]
    b_h = prm[:, 7:8]
    b_out = prm[:, 8:9]
    mask6 = prm[:, 9:10]
    tab = tab_ref[...]
    iot = jax.lax.broadcasted_iota(jnp.int32, (16, _CHK), 0).astype(jnp.float32)

    for j in range(_BLK // _CHK):
        lo, hi = j * _CHK, (j + 1) * _CHK
        xy = xyT_ref[:, lo:hi]
        cx = jnp.clip((xy[0:1, :] + 1.0) * 7.5, 0.0, 15.0)
        cy = jnp.clip((xy[1:2, :] + 1.0) * 7.5, 0.0, 15.0)
        ohx = jnp.maximum(0.0, 1.0 - jnp.abs(cx - iot))
        ohy = jnp.maximum(0.0, 1.0 - jnp.abs(cy - iot))
        w_interp = (ohy[:, None, :] * ohx[None, :, :]).reshape(256, _CHK)
        r = jnp.dot(tab, w_interp.astype(jnp.bfloat16),
                    preferred_element_type=jnp.float32)
        itp_ref[:, lo:hi] = r[0:32, :] + r[32:64, :]

    x = xyT_ref[0:1, :]
    y = xyT_ref[1:2, :]
    itp = itp_ref[...]
    g0 = itp[0:8, :]
    bb0 = itp[8:16, :]
    g1 = itp[16:24, :]
    bb1 = itp[24:32, :]
    h = x * w_in_x + y * w_in_y + b_in
    h = h * jax.nn.sigmoid(h)
    h = _ln_mod(h, mask6, lnw0, lnb0, g0, bb0)
    z = jnp.dot(whm_ref[...], h, preferred_element_type=jnp.float32) + b_h
    h = z * jax.nn.sigmoid(z)
    h = _ln_mod(h, mask6, lnw1, lnb1, g1, bb1)
    o = jnp.dot(wout_ref[...], h, preferred_element_type=jnp.float32) + b_out
    out_ref[...] = o[0:3, :]


def kernel(xy, gamma, beta, w_in, b_in, w_h, b_h, w_out, b_out, ln_w, ln_b):
    n = xy.shape[0]
    xyT = xy.T
    t = jnp.zeros((32, 256), jnp.float32)
    t = t.at[0:6].set(gamma[0].reshape(6, 256))
    t = t.at[8:14].set(beta[0].reshape(6, 256))
    t = t.at[16:22].set(gamma[1].reshape(6, 256))
    t = t.at[24:30].set(beta[1].reshape(6, 256))
    t_hi = t.astype(jnp.bfloat16)
    t_lo = (t - t_hi.astype(jnp.float32)).astype(jnp.bfloat16)
    tab = jnp.concatenate([t_hi, t_lo], axis=0)

    def col(v):
        return jnp.pad(v.astype(jnp.float32), (0, 8 - v.shape[0]))

    prm = jnp.stack([
        col(w_in[:, 0]), col(w_in[:, 1]), col(b_in),
        col(ln_w[0]), col(ln_b[0]), col(ln_w[1]), col(ln_b[1]),
        col(b_h[0]), col(b_out),
        jnp.array([1, 1, 1, 1, 1, 1, 0, 0], jnp.float32),
    ] + [jnp.zeros(8, jnp.float32)] * 6, axis=1)
    whm = jnp.zeros((8, 8), jnp.float32).at[0:6, 0:6].set(w_h[0])
    wout = jnp.zeros((8, 8), jnp.float32).at[0:3, 0:6].set(w_out)

    outT = pl.pallas_call(
        _body,
        out_shape=jax.ShapeDtypeStruct((3, n), jnp.float32),
        grid=(n // _BLK,),
        in_specs=[
            pl.BlockSpec((2, _BLK), lambda i: (0, i)),
            pl.BlockSpec((64, 256), lambda i: (0, 0)),
            pl.BlockSpec((8, 16), lambda i: (0, 0)),
            pl.BlockSpec((8, 8), lambda i: (0, 0)),
            pl.BlockSpec((8, 8), lambda i: (0, 0)),
        ],
        out_specs=pl.BlockSpec((3, _BLK), lambda i: (0, i)),
        scratch_shapes=[pltpu.VMEM((32, _BLK), jnp.float32)],
        compiler_params=pltpu.CompilerParams(
            dimension_semantics=("arbitrary",),
        ),
        name="camfield_fused",
    )(xyT, tab, prm, whm, wout)
    return outT.T


# speedup vs baseline: 1.7761x; 1.2256x over previous
import jax
import jax.numpy as jnp
from jax.experimental import pallas as pl
from jax.experimental.pallas import tpu as pltpu

_BLK = 32768
_CHK = 512
_EPS = 1e-5


def _ln_mod(h, mask6, red, lnw, lnb, g, bb):
    mu = jnp.sum(h, axis=0, keepdims=True) * (1.0 / 6.0)
    d = (h - mu) * mask6
    var = jnp.dot(red, d * d, preferred_element_type=jnp.float32)
    hn = d * jax.lax.rsqrt(var + _EPS)
    return g * (hn * lnw + lnb) + bb


def _body(xyT_ref, tab_ref, prm_ref, whm_ref, wout_ref, red_ref, out_ref,
          itp_ref):
    prm = prm_ref[...]
    w_in_x = prm[:, 0:1]
    w_in_y = prm[:, 1:2]
    b_in = prm[:, 2:3]
    lnw0 = prm[:, 3:4]
    lnb0 = prm[:, 4:5]
    lnw1 = prm[:, 5:6]
    lnb1 = prm[:, 6:7]
    b_h = prm[:, 7:8]
    b_out = prm[:, 8:9]
    mask6 = prm[:, 9:10]
    tab = tab_ref[...]
    iot = jax.lax.broadcasted_iota(jnp.int32, (16, _CHK), 0).astype(jnp.float32)

    for j in range(_BLK // _CHK):
        lo, hi = j * _CHK, (j + 1) * _CHK
        xy = xyT_ref[:, lo:hi]
        cx = jnp.clip((xy[0:1, :] + 1.0) * 7.5, 0.0, 15.0)
        cy = jnp.clip((xy[1:2, :] + 1.0) * 7.5, 0.0, 15.0)
        ohx = jnp.maximum(0.0, 1.0 - jnp.abs(cx - iot))
        ohy = jnp.maximum(0.0, 1.0 - jnp.abs(cy - iot))
        rows = []
        for yv in range(16):
            b = jnp.broadcast_to(ohy[yv:yv + 1, :], (8, _CHK))
            rows.append(b)
            rows.append(b)
        ohy_exp = jnp.concatenate(rows, axis=0)
        w_interp = ohy_exp * jnp.tile(ohx, (16, 1))
        r = jnp.dot(tab, w_interp.astype(jnp.bfloat16),
                    preferred_element_type=jnp.float32)
        itp_ref[:, lo:hi] = r[0:32, :] + r[32:64, :]

    x = xyT_ref[0:1, :]
    y = xyT_ref[1:2, :]
    itp = itp_ref[...]
    g0 = itp[0:8, :]
    bb0 = itp[8:16, :]
    g1 = itp[16:24, :]
    bb1 = itp[24:32, :]
    red = red_ref[...]
    h = x * w_in_x + y * w_in_y + b_in
    h = h * jax.nn.sigmoid(h)
    h = _ln_mod(h, mask6, red, lnw0, lnb0, g0, bb0)
    z = jnp.dot(whm_ref[...], h, preferred_element_type=jnp.float32) + b_h
    h = z * jax.nn.sigmoid(z)
    h = _ln_mod(h, mask6, red, lnw1, lnb1, g1, bb1)
    o = jnp.dot(wout_ref[...], h, preferred_element_type=jnp.float32) + b_out
    out_ref[...] = o[0:3, :]


def kernel(xy, gamma, beta, w_in, b_in, w_h, b_h, w_out, b_out, ln_w, ln_b):
    n = xy.shape[0]
    xyT = xy.T
    t = jnp.zeros((32, 256), jnp.float32)
    t = t.at[0:6].set(gamma[0].reshape(6, 256))
    t = t.at[8:14].set(beta[0].reshape(6, 256))
    t = t.at[16:22].set(gamma[1].reshape(6, 256))
    t = t.at[24:30].set(beta[1].reshape(6, 256))
    t_hi = t.astype(jnp.bfloat16)
    t_lo = (t - t_hi.astype(jnp.float32)).astype(jnp.bfloat16)
    tab = jnp.concatenate([t_hi, t_lo], axis=0)

    def col(v):
        return jnp.pad(v.astype(jnp.float32), (0, 8 - v.shape[0]))

    prm = jnp.stack([
        col(w_in[:, 0]), col(w_in[:, 1]), col(b_in),
        col(ln_w[0]), col(ln_b[0]), col(ln_w[1]), col(ln_b[1]),
        col(b_h[0]), col(b_out),
        jnp.array([1, 1, 1, 1, 1, 1, 0, 0], jnp.float32),
    ] + [jnp.zeros(8, jnp.float32)] * 6, axis=1)
    whm = jnp.zeros((8, 8), jnp.float32).at[0:6, 0:6].set(w_h[0])
    wout = jnp.zeros((8, 8), jnp.float32).at[0:3, 0:6].set(w_out)
    red = jnp.concatenate(
        [jnp.full((8, 6), 1.0 / 6.0, jnp.float32),
         jnp.zeros((8, 2), jnp.float32)], axis=1)

    outT = pl.pallas_call(
        _body,
        out_shape=jax.ShapeDtypeStruct((3, n), jnp.float32),
        grid=(n // _BLK,),
        in_specs=[
            pl.BlockSpec((2, _BLK), lambda i: (0, i)),
            pl.BlockSpec((64, 256), lambda i: (0, 0)),
            pl.BlockSpec((8, 16), lambda i: (0, 0)),
            pl.BlockSpec((8, 8), lambda i: (0, 0)),
            pl.BlockSpec((8, 8), lambda i: (0, 0)),
            pl.BlockSpec((8, 8), lambda i: (0, 0)),
        ],
        out_specs=pl.BlockSpec((3, _BLK), lambda i: (0, i)),
        scratch_shapes=[pltpu.VMEM((32, _BLK), jnp.float32)],
        compiler_params=pltpu.CompilerParams(
            dimension_semantics=("arbitrary",),
        ),
        name="camfield_fused",
    )(xyT, tab, prm, whm, wout, red)
    return outT.T


# speedup vs baseline: 1.8333x; 1.0322x over previous
import jax
import jax.numpy as jnp
from jax.experimental import pallas as pl
from jax.experimental.pallas import tpu as pltpu

_BLK = 32768
_CHK = 512
_EPS = 1e-5


def _ln_mod(h, mask6, red, lnw, lnb, g, bb):
    mu = jnp.sum(h, axis=0, keepdims=True) * (1.0 / 6.0)
    d = (h - mu) * mask6
    var = jnp.dot(red, d * d, preferred_element_type=jnp.float32)
    hn = d * jax.lax.rsqrt(var + _EPS)
    return g * (hn * lnw + lnb) + bb


def _body(xyT_ref, tab_ref, prm_ref, whm_ref, wout_ref, red_ref, out_ref,
          itp_ref):
    prm = prm_ref[...]
    w_in_x = prm[:, 0:1]
    w_in_y = prm[:, 1:2]
    b_in = prm[:, 2:3]
    lnw0 = prm[:, 3:4]
    lnb0 = prm[:, 4:5]
    lnw1 = prm[:, 5:6]
    lnb1 = prm[:, 6:7]
    b_h = prm[:, 7:8]
    b_out = prm[:, 8:9]
    mask6 = prm[:, 9:10]
    tab = tab_ref[...]
    iot = jax.lax.broadcasted_iota(jnp.int32, (16, _CHK), 0).astype(jnp.float32)

    for j in range(_BLK // _CHK):
        lo, hi = j * _CHK, (j + 1) * _CHK
        xy = xyT_ref[:, lo:hi]
        cx = jnp.clip((xy[0:1, :] + 1.0) * 7.5, 0.0, 15.0)
        cy = jnp.clip((xy[1:2, :] + 1.0) * 7.5, 0.0, 15.0)
        ohx = jnp.maximum(0.0, 1.0 - jnp.abs(cx - iot))
        ohy = jnp.maximum(0.0, 1.0 - jnp.abs(cy - iot))
        rows = []
        for yv in range(16):
            b = jnp.broadcast_to(ohy[yv:yv + 1, :], (8, _CHK))
            rows.append(b)
            rows.append(b)
        ohy_exp = jnp.concatenate(rows, axis=0)
        w_interp = ohy_exp * jnp.tile(ohx, (16, 1))
        itp_ref[:, lo:hi] = jnp.dot(tab, w_interp,
                                    preferred_element_type=jnp.float32)

    x = xyT_ref[0:1, :]
    y = xyT_ref[1:2, :]
    itp = itp_ref[...]
    g0 = itp[0:8, :]
    bb0 = itp[8:16, :]
    g1 = itp[16:24, :]
    bb1 = itp[24:32, :]
    red = red_ref[...]
    h = x * w_in_x + y * w_in_y + b_in
    h = h * jax.nn.sigmoid(h)
    h = _ln_mod(h, mask6, red, lnw0, lnb0, g0, bb0)
    z = jnp.dot(whm_ref[...], h, preferred_element_type=jnp.float32) + b_h
    h = z * jax.nn.sigmoid(z)
    h = _ln_mod(h, mask6, red, lnw1, lnb1, g1, bb1)
    o = jnp.dot(wout_ref[...], h, preferred_element_type=jnp.float32) + b_out
    out_ref[...] = o[0:3, :]


def kernel(xy, gamma, beta, w_in, b_in, w_h, b_h, w_out, b_out, ln_w, ln_b):
    n = xy.shape[0]
    xyT = xy.T
    t = jnp.zeros((32, 256), jnp.float32)
    t = t.at[0:6].set(gamma[0].reshape(6, 256))
    t = t.at[8:14].set(beta[0].reshape(6, 256))
    t = t.at[16:22].set(gamma[1].reshape(6, 256))
    t = t.at[24:30].set(beta[1].reshape(6, 256))
    tab = t

    def col(v):
        return jnp.pad(v.astype(jnp.float32), (0, 8 - v.shape[0]))

    prm = jnp.stack([
        col(w_in[:, 0]), col(w_in[:, 1]), col(b_in),
        col(ln_w[0]), col(ln_b[0]), col(ln_w[1]), col(ln_b[1]),
        col(b_h[0]), col(b_out),
        jnp.array([1, 1, 1, 1, 1, 1, 0, 0], jnp.float32),
    ] + [jnp.zeros(8, jnp.float32)] * 6, axis=1)
    whm = jnp.zeros((8, 8), jnp.float32).at[0:6, 0:6].set(w_h[0])
    wout = jnp.zeros((8, 8), jnp.float32).at[0:3, 0:6].set(w_out)
    red = jnp.concatenate(
        [jnp.full((8, 6), 1.0 / 6.0, jnp.float32),
         jnp.zeros((8, 2), jnp.float32)], axis=1)

    outT = pl.pallas_call(
        _body,
        out_shape=jax.ShapeDtypeStruct((3, n), jnp.float32),
        grid=(n // _BLK,),
        in_specs=[
            pl.BlockSpec((2, _BLK), lambda i: (0, i)),
            pl.BlockSpec((32, 256), lambda i: (0, 0)),
            pl.BlockSpec((8, 16), lambda i: (0, 0)),
            pl.BlockSpec((8, 8), lambda i: (0, 0)),
            pl.BlockSpec((8, 8), lambda i: (0, 0)),
            pl.BlockSpec((8, 8), lambda i: (0, 0)),
        ],
        out_specs=pl.BlockSpec((3, _BLK), lambda i: (0, i)),
        scratch_shapes=[pltpu.VMEM((32, _BLK), jnp.float32)],
        compiler_params=pltpu.CompilerParams(
            dimension_semantics=("arbitrary",),
        ),
        name="camfield_fused",
    )(xyT, tab, prm, whm, wout, red)
    return outT.T
